# baseline (speedup 1.0000x reference)
import functools

import jax
import jax.numpy as jnp
from jax import lax
from jax.experimental import pallas as pl
from jax.experimental.pallas import tpu as pltpu
from jax.experimental.pallas import tpu_sc as plsc

_VOCAB = 1000000
_DIM = 64
_BATCH = 16384
_NC = 2
_NS = 16
_LANES = 16
_NW = _NC * _NS
_BPW = _BATCH // _NW
_K = 128
_NCHUNK = _BPW // _K
_GROUPS = _K // _LANES


def _dot_body(center_hbm, context_hbm, w_in_hbm, w_out_hbm, out_hbm,
              idx_c, idx_x, rows_in, rows_out, out_v, sem_idx, sem_rows):
    wid = lax.axis_index("s") * _NC + lax.axis_index("c")
    base = wid * _BPW

    for c in range(_NCHUNK):
        pltpu.async_copy(center_hbm.at[pl.ds(base + c * _K, _K)],
                         idx_c.at[c], sem_idx)
        pltpu.async_copy(context_hbm.at[pl.ds(base + c * _K, _K)],
                         idx_x.at[c], sem_idx)
    for c in range(_NCHUNK):
        pltpu.make_async_copy(center_hbm.at[pl.ds(base + c * _K, _K)],
                              idx_c.at[c], sem_idx).wait()
        pltpu.make_async_copy(context_hbm.at[pl.ds(base + c * _K, _K)],
                              idx_x.at[c], sem_idx).wait()

    def fire(c):
        pltpu.async_copy(w_in_hbm.at[idx_c.at[c]], rows_in.at[c], sem_rows)
        pltpu.async_copy(w_out_hbm.at[idx_x.at[c]], rows_out.at[c], sem_rows)

    def drain(c):
        pltpu.make_async_copy(w_in_hbm.at[idx_c.at[c]], rows_in.at[c],
                              sem_rows).wait()
        pltpu.make_async_copy(w_out_hbm.at[idx_x.at[c]], rows_out.at[c],
                              sem_rows).wait()

    fire(0)
    for c in range(_NCHUNK):
        if c + 1 < _NCHUNK:
            fire(c + 1)
        drain(c)

        lane = lax.iota(jnp.int32, _LANES)

        def group(g, _):
            acc = jnp.zeros((_LANES,), jnp.float32)
            row0 = g * _LANES
            for j in range(_LANES):
                s = jnp.zeros((_LANES,), jnp.float32)
                for k in range(_DIM // _LANES):
                    a = rows_in[c, row0 + j, pl.ds(k * _LANES, _LANES)]
                    b = rows_out[c, row0 + j, pl.ds(k * _LANES, _LANES)]
                    s = s + a * b
                for sh in (8, 4, 2, 1):
                    s = s + s.at[lane ^ sh].get(mode="promise_in_bounds")
                acc = jnp.where(lane == j, s, acc)
            out_v[pl.ds(c * _K + row0, _LANES)] = acc
            return 0

        lax.fori_loop(0, _GROUPS, group, 0)

    pltpu.sync_copy(out_v, out_hbm.at[pl.ds(base, _BPW)])


@jax.jit
def kernel(center, context, W_in, W_out):
    mesh = plsc.VectorSubcoreMesh(core_axis_name="c", subcore_axis_name="s")
    run = pl.kernel(
        _dot_body,
        out_type=jax.ShapeDtypeStruct((_BATCH,), jnp.float32),
        mesh=mesh,
        scratch_types=[
            pltpu.VMEM((_NCHUNK, _K), jnp.int32),
            pltpu.VMEM((_NCHUNK, _K), jnp.int32),
            pltpu.VMEM((_NCHUNK, _K, _DIM), jnp.float32),
            pltpu.VMEM((_NCHUNK, _K, _DIM), jnp.float32),
            pltpu.VMEM((_BPW,), jnp.float32),
            pltpu.SemaphoreType.DMA,
            pltpu.SemaphoreType.DMA,
        ],
        compiler_params=pltpu.CompilerParams(use_tc_tiling_on_sc=False),
    )
    return run(center, context, W_in, W_out)
